# baseline (speedup 1.0000x reference)
import functools

import jax
from jax._src.config import enable_x64 as _x64_ctx
import jax.numpy as jnp
from jax import lax
from jax.experimental import pallas as pl
from jax.experimental.pallas import tpu as pltpu
from jax.experimental.pallas import tpu_sc as plsc

_ZCH = 1000000
_D = 32
_NC = 2
_NS = 16
_LANES = 16
_B = 16384
_H = 20
_PER_W = (_B * _H) // _NS
_ROWS = 128
_NJ = _PER_W // _ROWS
_NG = 8
_CG = _H
_C_HI = 435
_C_LO = 761


def _vconst(x):
    return jnp.full((_LANES,), x, dtype=jnp.int32)


def _hash16(v):
    t = lax.rem(v * _vconst(_C_HI), _vconst(_ZCH))
    return lax.rem(t * _vconst(1000) + v * _vconst(_C_LO), _vconst(_ZCH))


def _remap_body(ids_hbm, rmp_hbm, ids_v, idx_v):
    f = lax.axis_index("c")
    w = lax.axis_index("s")
    pltpu.sync_copy(ids_hbm.at[f, w], ids_v)

    @pl.loop(0, _NJ)
    def _hash_row(j):
        for k in range(_ROWS // _LANES):
            sl = pl.ds(k * _LANES, _LANES)
            idx_v[j, sl] = _hash16(ids_v[j, sl])

    pltpu.sync_copy(idx_v, rmp_hbm.at[f, w])


_sc_remap = functools.partial(
    pl.kernel,
    out_type=jax.ShapeDtypeStruct((2, _NS, _NJ, _ROWS), jnp.int32),
    mesh=plsc.VectorSubcoreMesh(
        core_axis_name="c", subcore_axis_name="s",
        num_cores=_NC, num_subcores=_NS),
    compiler_params=pltpu.CompilerParams(use_tc_tiling_on_sc=False),
    scratch_types=(
        pltpu.VMEM((_NJ, _ROWS), jnp.int32),
        pltpu.VMEM((_NJ, _ROWS), jnp.int32),
    ),
)(_remap_body)


def _gather_body(ids_hbm, t0_hbm, t1_hbm, out_hbm,
                 ids_g, idx_g, block, buf0, buf1, gsem0, gsem1, osem):
    f = lax.axis_index("c")
    w = lax.axis_index("s")
    iota = lax.iota(jnp.int32, _LANES)
    bufs = (buf0, buf1)
    gsems = (gsem0, gsem1)

    def run(tbl):
        def gstart(c, s):
            pltpu.async_copy(tbl.at[idx_g.at[c]], bufs[s], gsems[s])

        def gwait(s):
            pltpu.make_async_copy(tbl.at[idx_g.at[0]], bufs[s],
                                  gsems[s]).wait()

        @pl.loop(0, _NG)
        def _group(g):
            pltpu.sync_copy(ids_hbm.at[f, w, pl.ds(g * _CG, _CG)], ids_g)

            @pl.loop(0, _CG)
            def _hash_row(j):
                for k in range(_ROWS // _LANES):
                    sl = pl.ds(k * _LANES, _LANES)
                    idx_g[j, sl] = _hash16(ids_g[j, sl])

            gstart(0, 0)
            gstart(1, 1)

            @pl.when(g > 0)
            def _():
                for _h in range(_H):
                    pltpu.make_async_copy(block.at[0], out_hbm.at[f, 0, :, 0],
                                          osem).wait()

            @pl.loop(0, _CG, step=2)
            def _chunk(cb):
                for s in range(2):
                    c = cb + s
                    gwait(s)
                    buf = bufs[s]
                    for ib in range(_ROWS // _LANES):
                        l_vec = jnp.full(
                            (_LANES,), c * _ROWS + ib * _LANES,
                            jnp.int32) + iota
                        b_vec = lax.div(l_vec, _vconst(_H))
                        h_vec = l_vec - b_vec * _vconst(_H)
                        for d in range(_D):
                            vals = plsc.load_gather(
                                buf, [ib * _LANES + iota, _vconst(d)])
                            plsc.store_scatter(
                                block,
                                [h_vec, _vconst(d // 8),
                                 _vconst((d % 8) * _ROWS) + b_vec],
                                vals)

                    @pl.when(c + 2 < _CG)
                    def _():
                        gstart(c + 2, s)

            bt = w * _NG + g
            for h in range(_H):
                pltpu.async_copy(block.at[h], out_hbm.at[f, h, :, bt], osem)

        for _h in range(_H):
            pltpu.make_async_copy(block.at[0], out_hbm.at[f, 0, :, 0],
                                  osem).wait()

    @pl.when(f == 0)
    def _():
        run(t0_hbm)

    @pl.when(f == 1)
    def _():
        run(t1_hbm)


_sc_gather = functools.partial(
    pl.kernel,
    out_type=jax.ShapeDtypeStruct((2, _H, _D // 8, _B // _ROWS, 8 * _ROWS),
                                  jnp.float32),
    mesh=plsc.VectorSubcoreMesh(
        core_axis_name="c", subcore_axis_name="s",
        num_cores=_NC, num_subcores=_NS),
    compiler_params=pltpu.CompilerParams(
        use_tc_tiling_on_sc=False, needs_layout_passes=False),
    scratch_types=(
        pltpu.VMEM((_CG, _ROWS), jnp.int32),
        pltpu.VMEM((_CG, _ROWS), jnp.int32),
        pltpu.VMEM((_H, _D // 8, 8 * _ROWS), jnp.float32),
        pltpu.VMEM((_ROWS, _D), jnp.float32),
        pltpu.VMEM((_ROWS, _D), jnp.float32),
        pltpu.SemaphoreType.DMA,
        pltpu.SemaphoreType.DMA,
        pltpu.SemaphoreType.DMA,
    ),
)(_gather_body)


def kernel(indices, table_0, table_1):
    with _x64_ctx(False):
        ids32 = indices.astype(jnp.int32).reshape(2, _NS, _NJ, _ROWS)
        rmp = _sc_remap(ids32)
        o6 = _sc_gather(ids32, table_0, table_1)
    embeddings = (
        o6.reshape(2, _H, _D // 8, _B // _ROWS, 8, _ROWS)
        .transpose(0, 3, 5, 1, 2, 4)
        .reshape(2, _B, _H, _D)
    )
    remapped = rmp.astype(indices.dtype).reshape(2, _B, _H)
    return embeddings, remapped


# speedup vs baseline: 1.0853x; 1.0853x over previous
import functools

import jax
from jax._src.config import enable_x64 as _x64_ctx
import jax.numpy as jnp
from jax import lax
from jax.experimental import pallas as pl
from jax.experimental.pallas import tpu as pltpu
from jax.experimental.pallas import tpu_sc as plsc

_ZCH = 1000000
_D = 32
_NC = 2
_NS = 16
_LANES = 16
_B = 16384
_H = 20
_PER_W = (_B * _H) // _NS
_ROWS = 128
_NJ = _PER_W // _ROWS
_NG = 8
_CG = _H
_C_HI = 435
_C_LO = 761


def _vconst(x):
    return jnp.full((_LANES,), x, dtype=jnp.int32)


def _hash16(v):
    t = lax.rem(v * _vconst(_C_HI), _vconst(_ZCH))
    return lax.rem(t * _vconst(1000) + v * _vconst(_C_LO), _vconst(_ZCH))


def _remap_body(ids_hbm, rmp_hbm, ids_v, idx_v):
    f = lax.axis_index("c")
    w = lax.axis_index("s")
    pltpu.sync_copy(ids_hbm.at[f, w], ids_v)

    @pl.loop(0, _NJ)
    def _hash_row(j):
        for k in range(_ROWS // _LANES):
            sl = pl.ds(k * _LANES, _LANES)
            idx_v[j, sl] = _hash16(ids_v[j, sl])

    pltpu.sync_copy(idx_v, rmp_hbm.at[f, w])


_sc_remap = functools.partial(
    pl.kernel,
    out_type=jax.ShapeDtypeStruct((2, _NS, _NJ, _ROWS), jnp.int32),
    mesh=plsc.VectorSubcoreMesh(
        core_axis_name="c", subcore_axis_name="s",
        num_cores=_NC, num_subcores=_NS),
    compiler_params=pltpu.CompilerParams(use_tc_tiling_on_sc=False),
    scratch_types=(
        pltpu.VMEM((_NJ, _ROWS), jnp.int32),
        pltpu.VMEM((_NJ, _ROWS), jnp.int32),
    ),
)(_remap_body)


def _gather_body(ids_hbm, t0_hbm, t1_hbm, out_hbm,
                 ids_g, idx_g, buf0, buf1, mini0, mini1,
                 gsem0, gsem1, osem0, osem1):
    f = lax.axis_index("c")
    w = lax.axis_index("s")
    iota = lax.iota(jnp.int32, _LANES)
    bufs = (buf0, buf1)
    minis = (mini0, mini1)
    gsems = (gsem0, gsem1)
    osems = (osem0, osem1)

    def run(tbl):
        def gstart(c, s):
            pltpu.async_copy(tbl.at[idx_g.at[c]], bufs[s], gsems[s])

        def gwait(s):
            pltpu.make_async_copy(tbl.at[idx_g.at[0]], bufs[s],
                                  gsems[s]).wait()

        def owait(s):
            pltpu.make_async_copy(minis[s], out_hbm.at[f, 0, :, 0],
                                  osems[s]).wait()

        @pl.loop(0, _NG)
        def _group(g):
            pltpu.sync_copy(ids_hbm.at[f, w, pl.ds(g * _CG * _ROWS,
                                                   _CG * _ROWS)], ids_g)

            @pl.loop(0, _CG)
            def _permrow(h):
                for kb in range(_ROWS // _LANES):
                    src = ((kb * _LANES + iota) * _vconst(_H)
                           + jnp.full((_LANES,), h, jnp.int32))
                    v = plsc.load_gather(ids_g, [src])
                    idx_g[h, pl.ds(kb * _LANES, _LANES)] = _hash16(v)

            gstart(0, 0)
            gstart(1, 1)

            @pl.loop(0, _CG, step=2)
            def _chunk(cb):
                for s in range(2):
                    c = cb + s
                    gwait(s)

                    @pl.when((g > 0) | (c >= 2))
                    def _():
                        owait(s)

                    buf = bufs[s]
                    mini = minis[s]
                    for kb in range(_ROWS // _LANES):
                        row_vec = kb * _LANES + iota
                        for dt in range(_D // 8):
                            for ds in range(8):
                                d = dt * 8 + ds
                                v = plsc.load_gather(
                                    buf, [row_vec, _vconst(d)])
                                mini[dt, pl.ds(ds * _ROWS + kb * _LANES,
                                               _LANES)] = v

                    @pl.when(c + 2 < _CG)
                    def _():
                        gstart(c + 2, s)

                    pltpu.async_copy(mini, out_hbm.at[f, c, :, w * _NG + g],
                                     osems[s])

        owait(0)
        owait(1)

    @pl.when(f == 0)
    def _():
        run(t0_hbm)

    @pl.when(f == 1)
    def _():
        run(t1_hbm)


_sc_gather = functools.partial(
    pl.kernel,
    out_type=jax.ShapeDtypeStruct((2, _H, _D // 8, _B // _ROWS, 8 * _ROWS),
                                  jnp.float32),
    mesh=plsc.VectorSubcoreMesh(
        core_axis_name="c", subcore_axis_name="s",
        num_cores=_NC, num_subcores=_NS),
    compiler_params=pltpu.CompilerParams(
        use_tc_tiling_on_sc=False, needs_layout_passes=False),
    scratch_types=(
        pltpu.VMEM((_CG * _ROWS,), jnp.int32),
        pltpu.VMEM((_CG, _ROWS), jnp.int32),
        pltpu.VMEM((_ROWS, _D), jnp.float32),
        pltpu.VMEM((_ROWS, _D), jnp.float32),
        pltpu.VMEM((_D // 8, 8 * _ROWS), jnp.float32),
        pltpu.VMEM((_D // 8, 8 * _ROWS), jnp.float32),
        pltpu.SemaphoreType.DMA,
        pltpu.SemaphoreType.DMA,
        pltpu.SemaphoreType.DMA,
        pltpu.SemaphoreType.DMA,
    ),
)(_gather_body)


def kernel(indices, table_0, table_1):
    with _x64_ctx(False):
        ids32 = indices.astype(jnp.int32).reshape(2, _NS, _NJ, _ROWS)
        rmp = _sc_remap(ids32)
        o6 = _sc_gather(ids32.reshape(2, _NS, _PER_W), table_0, table_1)
    embeddings = (
        o6.reshape(2, _H, _D // 8, _B // _ROWS, 8, _ROWS)
        .transpose(0, 3, 5, 1, 2, 4)
        .reshape(2, _B, _H, _D)
    )
    remapped = rmp.astype(indices.dtype).reshape(2, _B, _H)
    return embeddings, remapped


# speedup vs baseline: 1.2180x; 1.1223x over previous
import functools

import numpy as np

import jax
from jax._src.config import enable_x64 as _x64_ctx
import jax.numpy as jnp
from jax import lax
from jax.experimental import pallas as pl
from jax.experimental.pallas import tpu as pltpu
from jax.experimental.pallas import tpu_sc as plsc

_ZCH = 1000000
_D = 32
_NC = 2
_NS = 16
_LANES = 16
_B = 16384
_H = 20
_PER_W = (_B * _H) // _NS
_ROWS = 128
_NJ = _PER_W // _ROWS
_NG = 8
_CG = _H
_C_HI = 435
_C_LO = 761


def _vconst(x):
    return jnp.full((_LANES,), x, dtype=jnp.int32)


def _hash16(v):
    t = lax.rem(v * _vconst(_C_HI), _vconst(_ZCH))
    return lax.rem(t * _vconst(1000) + v * _vconst(_C_LO), _vconst(_ZCH))


def _remap_body(ids_hbm, rmp_hbm, ids_v, idx_v):
    f = lax.axis_index("c")
    w = lax.axis_index("s")
    pltpu.sync_copy(ids_hbm.at[f, w], ids_v)

    @pl.loop(0, _NJ)
    def _hash_row(j):
        for k in range(_ROWS // _LANES):
            sl = pl.ds(k * _LANES, _LANES)
            idx_v[j, sl] = _hash16(ids_v[j, sl])

    pltpu.sync_copy(idx_v, rmp_hbm.at[f, w])


_sc_remap = functools.partial(
    pl.kernel,
    out_type=jax.ShapeDtypeStruct((2, _NS, _NJ, _ROWS), jnp.int32),
    mesh=plsc.VectorSubcoreMesh(
        core_axis_name="c", subcore_axis_name="s",
        num_cores=_NC, num_subcores=_NS),
    compiler_params=pltpu.CompilerParams(use_tc_tiling_on_sc=False),
    scratch_types=(
        pltpu.VMEM((_NJ, _ROWS), jnp.int32),
        pltpu.VMEM((_NJ, _ROWS), jnp.int32),
    ),
)(_remap_body)


def _gather_body(ids_hbm, t0_hbm, t1_hbm, out_hbm,
                 ids_g, idx_g, buf0, buf1, mini0, mini1,
                 gsem0, gsem1, osem0, osem1):
    f = lax.axis_index("c")
    w = lax.axis_index("s")
    iota = lax.iota(jnp.int32, _LANES)
    bufs = (buf0, buf1)
    minis = (mini0, mini1)
    gsems = (gsem0, gsem1)
    osems = (osem0, osem1)

    def run(tbl):
        def gstart(c, s):
            pltpu.async_copy(tbl.at[idx_g.at[c]], bufs[s], gsems[s])

        def gwait(s):
            pltpu.make_async_copy(tbl.at[idx_g.at[0]], bufs[s],
                                  gsems[s]).wait()

        def owait(s):
            pltpu.make_async_copy(minis[s], out_hbm.at[f, 0, :, 0],
                                  osems[s]).wait()

        @pl.loop(0, _NG)
        def _group(g):
            pltpu.sync_copy(ids_hbm.at[f, w, pl.ds(g * _CG * _ROWS,
                                                   _CG * _ROWS)], ids_g)

            @pl.loop(0, _CG)
            def _permrow(h):
                for kb in range(_ROWS // _LANES):
                    src = ((kb * _LANES + iota) * _vconst(_H)
                           + jnp.full((_LANES,), h, jnp.int32))
                    v = plsc.load_gather(ids_g, [src])
                    idx_g[h, pl.ds(kb * _LANES, _LANES)] = _hash16(v)

            gstart(0, 0)
            gstart(1, 1)

            @pl.loop(0, _CG, step=2)
            def _chunk(cb):
                for s in range(2):
                    c = cb + s
                    gwait(s)

                    @pl.when((g > 0) | (c >= 2))
                    def _():
                        owait(s)

                    buf = bufs[s]
                    mini = minis[s]
                    for d0 in range(0, _D, _LANES):
                        for k in range(_LANES):
                            rot = lax.rem(iota + _vconst(k), _vconst(_LANES))
                            drot = rot + _vconst(d0)
                            dt_vec = lax.shift_right_logical(
                                drot, _vconst(3))
                            pcol = lax.shift_left(
                                lax.bitwise_and(drot, _vconst(7)),
                                _vconst(7)) + iota
                            for kb in range(_ROWS // _LANES):
                                b0 = kb * _LANES
                                v = plsc.load_gather(
                                    buf, [iota + _vconst(b0), drot])
                                plsc.store_scatter(
                                    mini, [dt_vec, pcol + _vconst(b0)], v)

                    @pl.when(c + 2 < _CG)
                    def _():
                        gstart(c + 2, s)

                    pltpu.async_copy(mini, out_hbm.at[f, c, :, w * _NG + g],
                                     osems[s])

        owait(0)
        owait(1)

    @pl.when(f == 0)
    def _():
        run(t0_hbm)

    @pl.when(f == 1)
    def _():
        run(t1_hbm)


_sc_gather = functools.partial(
    pl.kernel,
    out_type=jax.ShapeDtypeStruct((2, _H, _D // 8, _B // _ROWS, 8 * _ROWS),
                                  jnp.float32),
    mesh=plsc.VectorSubcoreMesh(
        core_axis_name="c", subcore_axis_name="s",
        num_cores=_NC, num_subcores=_NS),
    compiler_params=pltpu.CompilerParams(
        use_tc_tiling_on_sc=False, needs_layout_passes=False),
    scratch_types=(
        pltpu.VMEM((_CG * _ROWS,), jnp.int32),
        pltpu.VMEM((_CG, _ROWS), jnp.int32),
        pltpu.VMEM((_ROWS, _D), jnp.float32),
        pltpu.VMEM((_ROWS, _D), jnp.float32),
        pltpu.VMEM((_D // 8, 8 * _ROWS), jnp.float32),
        pltpu.VMEM((_D // 8, 8 * _ROWS), jnp.float32),
        pltpu.SemaphoreType.DMA,
        pltpu.SemaphoreType.DMA,
        pltpu.SemaphoreType.DMA,
        pltpu.SemaphoreType.DMA,
    ),
)(_gather_body)


def kernel(indices, table_0, table_1):
    with _x64_ctx(False):
        ids32 = indices.astype(jnp.int32).reshape(2, _NS, _NJ, _ROWS)
        rmp = _sc_remap(ids32)
        o6 = _sc_gather(ids32.reshape(2, _NS, _PER_W), table_0, table_1)
    embeddings = (
        o6.reshape(2, _H, _D // 8, _B // _ROWS, 8, _ROWS)
        .transpose(0, 3, 5, 1, 2, 4)
        .reshape(2, _B, _H, _D)
    )
    remapped = rmp.astype(indices.dtype).reshape(2, _B, _H)
    return embeddings, remapped


# speedup vs baseline: 1.2958x; 1.0639x over previous
import functools

import numpy as np

import jax
from jax._src.config import enable_x64 as _x64_ctx
import jax.numpy as jnp
from jax import lax
from jax.experimental import pallas as pl
from jax.experimental.pallas import tpu as pltpu
from jax.experimental.pallas import tpu_sc as plsc

_ZCH = 1000000
_D = 32
_NC = 2
_NS = 16
_LANES = 16
_B = 16384
_H = 20
_PER_W = (_B * _H) // _NS
_ROWS = 128
_NJ = _PER_W // _ROWS
_NG = 8
_CG = _H
_C_HI = 435
_C_LO = 761


def _vconst(x):
    return jnp.full((_LANES,), x, dtype=jnp.int32)


def _hash16(v):
    t = lax.rem(v * _vconst(_C_HI), _vconst(_ZCH))
    return lax.rem(t * _vconst(1000) + v * _vconst(_C_LO), _vconst(_ZCH))


def _remap_body(ids_hbm, rmp_hbm, ids_v, idx_v):
    f = lax.axis_index("c")
    w = lax.axis_index("s")
    pltpu.sync_copy(ids_hbm.at[f, w], ids_v)

    @pl.loop(0, _NJ)
    def _hash_row(j):
        for k in range(_ROWS // _LANES):
            sl = pl.ds(k * _LANES, _LANES)
            idx_v[j, sl] = _hash16(ids_v[j, sl])

    pltpu.sync_copy(idx_v, rmp_hbm.at[f, w])


_sc_remap = functools.partial(
    pl.kernel,
    out_type=jax.ShapeDtypeStruct((2, _NS, _NJ, _ROWS), jnp.int32),
    mesh=plsc.VectorSubcoreMesh(
        core_axis_name="c", subcore_axis_name="s",
        num_cores=_NC, num_subcores=_NS),
    compiler_params=pltpu.CompilerParams(use_tc_tiling_on_sc=False),
    scratch_types=(
        pltpu.VMEM((_NJ, _ROWS), jnp.int32),
        pltpu.VMEM((_NJ, _ROWS), jnp.int32),
    ),
)(_remap_body)


def _gather_body(ids_hbm, tbl, out_hbm,
                 ids_g, idx_g, buf0, buf1, mini0, mini1,
                 gsem0, gsem1, osem0, osem1):
    wid = lax.axis_index("c") * _NS + lax.axis_index("s")
    iota = lax.iota(jnp.int32, _LANES)
    bufs = (buf0, buf1)
    minis = (mini0, mini1)
    gsems = (gsem0, gsem1)
    osems = (osem0, osem1)

    if True:
        def gstart(c, s):
            pltpu.async_copy(tbl.at[idx_g.at[c]], bufs[s], gsems[s])

        def gwait(s):
            pltpu.make_async_copy(tbl.at[idx_g.at[0]], bufs[s],
                                  gsems[s]).wait()

        def owait(s):
            pltpu.make_async_copy(minis[s], out_hbm.at[0, :, 0],
                                  osems[s]).wait()

        @pl.loop(0, _NG // 2)
        def _group(g):
            pltpu.sync_copy(
                ids_hbm.at[pl.ds(wid * (_PER_W // 2) + g * _CG * _ROWS,
                                 _CG * _ROWS)], ids_g)

            @pl.loop(0, _CG)
            def _permrow(h):
                for kb in range(_ROWS // _LANES):
                    src = ((kb * _LANES + iota) * _vconst(_H)
                           + jnp.full((_LANES,), h, jnp.int32))
                    v = plsc.load_gather(ids_g, [src])
                    idx_g[h, pl.ds(kb * _LANES, _LANES)] = _hash16(v)

            gstart(0, 0)
            gstart(1, 1)

            @pl.loop(0, _CG, step=2)
            def _chunk(cb):
                for s in range(2):
                    c = cb + s
                    gwait(s)

                    @pl.when((g > 0) | (c >= 2))
                    def _():
                        owait(s)

                    buf = bufs[s]
                    mini = minis[s]
                    for d0 in range(0, _D, _LANES):
                        for k in range(_LANES):
                            rot = lax.rem(iota + _vconst(k), _vconst(_LANES))
                            drot = rot + _vconst(d0)
                            dt_vec = lax.shift_right_logical(
                                drot, _vconst(3))
                            pcol = lax.shift_left(
                                lax.bitwise_and(drot, _vconst(7)),
                                _vconst(7)) + iota
                            for kb in range(_ROWS // _LANES):
                                b0 = kb * _LANES
                                v = plsc.load_gather(
                                    buf, [iota + _vconst(b0), drot])
                                plsc.store_scatter(
                                    mini, [dt_vec, pcol + _vconst(b0)], v)

                    @pl.when(c + 2 < _CG)
                    def _():
                        gstart(c + 2, s)

                    pltpu.async_copy(
                        mini,
                        out_hbm.at[c, :, wid * (_NG // 2) + g],
                        osems[s])

        owait(0)
        owait(1)


_sc_gather = functools.partial(
    pl.kernel,
    out_type=jax.ShapeDtypeStruct((_H, _D // 8, _B // _ROWS, 8 * _ROWS),
                                  jnp.float32),
    mesh=plsc.VectorSubcoreMesh(
        core_axis_name="c", subcore_axis_name="s",
        num_cores=_NC, num_subcores=_NS),
    compiler_params=pltpu.CompilerParams(
        use_tc_tiling_on_sc=False, needs_layout_passes=False),
    scratch_types=(
        pltpu.VMEM((_CG * _ROWS,), jnp.int32),
        pltpu.VMEM((_CG, _ROWS), jnp.int32),
        pltpu.VMEM((_ROWS, _D), jnp.float32),
        pltpu.VMEM((_ROWS, _D), jnp.float32),
        pltpu.VMEM((_D // 8, 8 * _ROWS), jnp.float32),
        pltpu.VMEM((_D // 8, 8 * _ROWS), jnp.float32),
        pltpu.SemaphoreType.DMA,
        pltpu.SemaphoreType.DMA,
        pltpu.SemaphoreType.DMA,
        pltpu.SemaphoreType.DMA,
    ),
)(_gather_body)


def kernel(indices, table_0, table_1):
    with _x64_ctx(False):
        ids32 = indices.astype(jnp.int32).reshape(2, _NS, _NJ, _ROWS)
        rmp = _sc_remap(ids32)
        ids_flat = ids32.reshape(2, _NS * _PER_W)
        o0 = _sc_gather(ids_flat[0], table_0)
        o1 = _sc_gather(ids_flat[1], table_1)
        o6 = jnp.stack([o0, o1])
    embeddings = (
        o6.reshape(2, _H, _D // 8, _B // _ROWS, 8, _ROWS)
        .transpose(0, 3, 5, 1, 2, 4)
        .reshape(2, _B, _H, _D)
    )
    remapped = rmp.astype(indices.dtype).reshape(2, _B, _H)
    return embeddings, remapped
